# baseline (speedup 1.0000x reference)
import functools

import jax
import jax.numpy as jnp
from jax import lax
from jax.experimental import pallas as pl
from jax.experimental.pallas import tpu as pltpu
from jax.experimental.pallas import tpu_sc as plsc


@functools.partial(jax.jit, static_argnames=("n", "blk_w", "blk_c", "iters"))
def _lookup_t(x_t, w2, *, n, blk_w, blk_c, iters):
    mesh = plsc.VectorSubcoreMesh(core_axis_name="c", subcore_axis_name="s")
    info = plsc.get_sparse_core_info()
    nc = info.num_cores
    sub_x = blk_c * 128
    sub_o = blk_c * 512

    @functools.partial(
        pl.kernel,
        mesh=mesh,
        out_type=jax.ShapeDtypeStruct((4 * n,), jnp.float32),
        scratch_types=[
            pltpu.VMEM((sub_x,), jnp.int32),
            pltpu.VMEM((sub_x,), jnp.int32),
            pltpu.VMEM((sub_o,), jnp.float32),
            pltpu.VMEM((sub_o,), jnp.float32),
            pltpu.VMEM((16, 16), jnp.float32),
            pltpu.SemaphoreType.DMA,
            pltpu.SemaphoreType.DMA,
            pltpu.SemaphoreType.DMA,
        ],
        compiler_params=pltpu.CompilerParams(needs_layout_passes=False),
    )
    def k(
        x_hbm,
        w_hbm,
        out_hbm,
        x_v0,
        x_v1,
        out_v0,
        out_v1,
        tbl_v,
        in_sem,
        out_sem0,
        out_sem1,
    ):
        wid = lax.axis_index("s") * nc + lax.axis_index("c")
        xbase = wid * blk_w * 128
        obase = wid * blk_w * 512

        pltpu.sync_copy(w_hbm, tbl_v)
        io = lax.iota(jnp.int32, 16)
        x_vs = (x_v0, x_v1)
        out_vs = (out_v0, out_v1)
        out_sems = (out_sem0, out_sem1)

        def start_in(it, slot):
            off = pl.multiple_of(xbase + it * sub_x, 8)
            pltpu.async_copy(x_hbm.at[pl.ds(off, sub_x)], x_vs[slot], in_sem)

        def wait_in(slot):
            pltpu.make_async_copy(
                x_hbm.at[pl.ds(0, sub_x)], x_vs[slot], in_sem
            ).wait()

        def wait_out(slot):
            pltpu.make_async_copy(
                out_vs[slot], out_hbm.at[pl.ds(0, sub_o)], out_sems[slot]
            ).wait()

        start_in(0, 0)

        def pair_body(it2, _):
            for slot in (0, 1):
                it = 2 * it2 + slot
                x_v = x_vs[slot]
                out_v = out_vs[slot]
                wait_in(slot)

                @pl.when(it + 1 < iters)
                def _():
                    start_in(it + 1, 1 - slot)

                @pl.when(it2 >= 1)
                def _():
                    wait_out(slot)

                @plsc.parallel_loop(0, blk_c, unroll=4)
                def blk_body(blk):
                    xb = blk * 128
                    ob = blk * 512
                    tix = [
                        lax.shift_left(x_v[pl.ds(xb + 16 * v, 16)], 2)
                        for v in range(8)
                    ]
                    for kk in range(4):
                        os = [
                            plsc.load_gather(tbl_v, [tix[v] + kk, io])
                            for v in range(8)
                        ]
                        for v in range(8):
                            out_v[pl.ds(ob + 128 * kk + 16 * v, 16)] = os[v]

                pltpu.async_copy(
                    out_v,
                    out_hbm.at[pl.ds(obase + it * sub_o, sub_o)],
                    out_sems[slot],
                )
            return 0

        lax.fori_loop(0, iters // 2, pair_body, 0)
        wait_out(0)
        wait_out(1)

    return k(x_t, w2)


def kernel(x, emb_weight):
    b, t = x.shape
    n = b * t
    x_t = x.T.reshape(n).astype(jnp.int32)
    w_flat = emb_weight.reshape(16).astype(jnp.float32)
    w2 = jnp.tile(w_flat[:, None], (1, 16))

    nw = 32
    n_blk = n // 128
    blk_w = n_blk // nw
    assert blk_w * nw == n_blk and (b % 128) == 0
    blk_c = max(
        c for c in range(2, 81, 2) if blk_w % c == 0 and (blk_w // c) % 2 == 0
    )
    iters = blk_w // blk_c

    y_flat = _lookup_t(x_t, w2, n=n, blk_w=blk_w, blk_c=blk_c, iters=iters)
    z = y_flat.reshape(t, b // 128, 4, 128)
    return z.transpose(1, 3, 0, 2).reshape(b, t, 4)


# speedup vs baseline: 1.5893x; 1.5893x over previous
import functools

import jax
import jax.numpy as jnp
from jax import lax
from jax.experimental import pallas as pl
from jax.experimental.pallas import tpu as pltpu
from jax.experimental.pallas import tpu_sc as plsc

_NB = 64


@functools.partial(jax.jit, static_argnames=("b", "t"))
def _lookup_t(xq, w2, *, b, t):
    mesh = plsc.VectorSubcoreMesh(core_axis_name="c", subcore_axis_name="s")
    info = plsc.get_sparse_core_info()
    nc = info.num_cores
    nw = nc * info.num_subcores
    n = b * t
    bt = b // 128
    chunks_per_j = bt // _NB
    sub_o = _NB * 512

    @functools.partial(
        pl.kernel,
        mesh=mesh,
        out_type=jax.ShapeDtypeStruct((4 * n,), jnp.float32),
        scratch_types=[
            pltpu.VMEM((_NB, 128), jnp.int32),
            pltpu.VMEM((_NB, 128), jnp.int32),
            pltpu.VMEM((sub_o,), jnp.float32),
            pltpu.VMEM((sub_o,), jnp.float32),
            pltpu.VMEM((16, 16), jnp.float32),
            pltpu.SemaphoreType.DMA,
            pltpu.SemaphoreType.DMA,
            pltpu.SemaphoreType.DMA,
        ],
        compiler_params=pltpu.CompilerParams(needs_layout_passes=False),
    )
    def k(
        x_hbm,
        w_hbm,
        out_hbm,
        x_v0,
        x_v1,
        out_v0,
        out_v1,
        tbl_v,
        in_sem,
        out_sem0,
        out_sem1,
    ):
        wid = lax.axis_index("s") * nc + lax.axis_index("c")
        j_lo = (wid * t) // nw
        j_hi = ((wid + 1) * t) // nw
        steps = (j_hi - j_lo) * chunks_per_j

        pltpu.sync_copy(w_hbm, tbl_v)
        io = lax.iota(jnp.int32, 16)
        x_vs = (x_v0, x_v1)
        out_vs = (out_v0, out_v1)
        out_sems = (out_sem0, out_sem1)

        def step_coords(step):
            j = j_lo + step // chunks_per_j
            b0 = (step % chunks_per_j) * _NB
            return j, b0

        def start_in(step, slot):
            j, b0 = step_coords(step)
            pltpu.async_copy(
                x_hbm.at[j // 8, pl.ds(pl.multiple_of(b0, 8), _NB), j % 8],
                x_vs[slot],
                in_sem,
            )

        def wait_in(slot):
            pltpu.make_async_copy(
                x_hbm.at[0, pl.ds(0, _NB), 0], x_vs[slot], in_sem
            ).wait()

        def wait_out(slot):
            pltpu.make_async_copy(
                out_vs[slot], out_hbm.at[pl.ds(0, sub_o)], out_sems[slot]
            ).wait()

        start_in(0, 0)

        def pair_body(it2, _):
            for slot in (0, 1):
                step = 2 * it2 + slot
                x_v = x_vs[slot]
                out_v = out_vs[slot]

                @pl.when(step < steps)
                def _():
                    wait_in(slot)

                    @pl.when(step + 1 < steps)
                    def _():
                        start_in(step + 1, 1 - slot)

                    @pl.when(it2 >= 1)
                    def _():
                        wait_out(slot)

                    @plsc.parallel_loop(0, _NB, unroll=2)
                    def blk_body(blk):
                        ob = blk * 512
                        tix = [
                            lax.shift_left(x_v[blk, pl.ds(16 * v, 16)], 2)
                            for v in range(8)
                        ]
                        for kk in range(4):
                            os = [
                                plsc.load_gather(tbl_v, [tix[v] + kk, io])
                                for v in range(8)
                            ]
                            for v in range(8):
                                out_v[pl.ds(ob + 128 * kk + 16 * v, 16)] = (
                                    os[v]
                                )

                    j, b0 = step_coords(step)
                    off = pl.multiple_of((j * bt + b0) * 512, 8)
                    pltpu.async_copy(
                        out_v, out_hbm.at[pl.ds(off, sub_o)], out_sems[slot]
                    )
            return 0

        lax.fori_loop(0, (steps + 1) // 2, pair_body, 0)
        wait_out(0)
        wait_out(1)

    return k(xq, w2)


def kernel(x, emb_weight):
    b, t = x.shape
    assert b % 128 == 0 and t % 8 == 0
    xq = (
        x.astype(jnp.int32)
        .reshape(b // 128, 128, t // 8, 8)
        .transpose(2, 0, 3, 1)
    )
    w_flat = emb_weight.reshape(16).astype(jnp.float32)
    w2 = jnp.tile(w_flat[:, None], (1, 16))

    y_flat = _lookup_t(xq, w2, b=b, t=t)
    z = y_flat.reshape(t, b // 128, 4, 128)
    return z.transpose(1, 3, 0, 2).reshape(b, t, 4)


# speedup vs baseline: 1.6591x; 1.0439x over previous
import functools

import jax
import jax.numpy as jnp
from jax import lax
from jax.experimental import pallas as pl
from jax.experimental.pallas import tpu as pltpu
from jax.experimental.pallas import tpu_sc as plsc

_NB = 64


@functools.partial(jax.jit, static_argnames=("b", "t"))
def _lookup_t(xq, w2, *, b, t):
    mesh = plsc.VectorSubcoreMesh(core_axis_name="c", subcore_axis_name="s")
    info = plsc.get_sparse_core_info()
    nc = info.num_cores
    nw = nc * info.num_subcores
    n = b * t
    bt = b // 128
    chunks_per_j = bt // _NB
    sub_o = _NB * 512

    @functools.partial(
        pl.kernel,
        mesh=mesh,
        out_type=jax.ShapeDtypeStruct((4 * n,), jnp.float32),
        scratch_types=[
            pltpu.VMEM((_NB, 128), jnp.int32),
            pltpu.VMEM((_NB, 128), jnp.int32),
            pltpu.VMEM((sub_o,), jnp.float32),
            pltpu.VMEM((sub_o,), jnp.float32),
            pltpu.VMEM((16, 16), jnp.float32),
            pltpu.SemaphoreType.DMA,
            pltpu.SemaphoreType.DMA,
            pltpu.SemaphoreType.DMA,
        ],
        compiler_params=pltpu.CompilerParams(needs_layout_passes=False),
    )
    def k(
        x_hbm,
        w_hbm,
        out_hbm,
        x_v0,
        x_v1,
        out_v0,
        out_v1,
        tbl_v,
        in_sem,
        out_sem0,
        out_sem1,
    ):
        wid = lax.axis_index("s") * nc + lax.axis_index("c")
        n_chunks = t * chunks_per_j
        q_lo = (wid * n_chunks) // nw
        q_hi = ((wid + 1) * n_chunks) // nw
        steps = q_hi - q_lo

        pltpu.sync_copy(w_hbm, tbl_v)
        io = lax.iota(jnp.int32, 16)
        x_vs = (x_v0, x_v1)
        out_vs = (out_v0, out_v1)
        out_sems = (out_sem0, out_sem1)

        def step_coords(step):
            q = q_lo + step
            j = q // chunks_per_j
            b0 = (q % chunks_per_j) * _NB
            return j, b0

        def start_in(step, slot):
            j, b0 = step_coords(step)
            pltpu.async_copy(
                x_hbm.at[j // 8, pl.ds(pl.multiple_of(b0, 8), _NB), j % 8],
                x_vs[slot],
                in_sem,
            )

        def wait_in(slot):
            pltpu.make_async_copy(
                x_hbm.at[0, pl.ds(0, _NB), 0], x_vs[slot], in_sem
            ).wait()

        def wait_out(slot):
            pltpu.make_async_copy(
                out_vs[slot], out_hbm.at[pl.ds(0, sub_o)], out_sems[slot]
            ).wait()

        start_in(0, 0)

        def pair_body(it2, _):
            for slot in (0, 1):
                step = 2 * it2 + slot
                x_v = x_vs[slot]
                out_v = out_vs[slot]

                @pl.when(step < steps)
                def _():
                    wait_in(slot)

                    @pl.when(step + 1 < steps)
                    def _():
                        start_in(step + 1, 1 - slot)

                    @pl.when(it2 >= 1)
                    def _():
                        wait_out(slot)

                    @plsc.parallel_loop(0, _NB, unroll=2)
                    def blk_body(blk):
                        ob = blk * 512
                        tix = [
                            lax.shift_left(x_v[blk, pl.ds(16 * v, 16)], 2)
                            for v in range(8)
                        ]
                        for kk in range(4):
                            os = [
                                plsc.load_gather(tbl_v, [tix[v] + kk, io])
                                for v in range(8)
                            ]
                            for v in range(8):
                                out_v[pl.ds(ob + 128 * kk + 16 * v, 16)] = (
                                    os[v]
                                )

                    j, b0 = step_coords(step)
                    off = pl.multiple_of((j * bt + b0) * 512, 8)
                    pltpu.async_copy(
                        out_v, out_hbm.at[pl.ds(off, sub_o)], out_sems[slot]
                    )
            return 0

        lax.fori_loop(0, (steps + 1) // 2, pair_body, 0)
        wait_out(0)
        wait_out(1)

    return k(xq, w2)


def kernel(x, emb_weight):
    b, t = x.shape
    assert b % 128 == 0 and t % 8 == 0
    xq = (
        x.astype(jnp.int32)
        .reshape(b // 128, 128, t // 8, 8)
        .transpose(2, 0, 3, 1)
    )
    w_flat = emb_weight.reshape(16).astype(jnp.float32)
    w2 = jnp.tile(w_flat[:, None], (1, 16))

    y_flat = _lookup_t(xq, w2, b=b, t=t)
    z = y_flat.reshape(t, b // 128, 4, 128)
    return z.transpose(1, 3, 0, 2).reshape(b, t, 4)


# speedup vs baseline: 1.7767x; 1.0709x over previous
import functools

import jax
import jax.numpy as jnp
from jax import lax
from jax.experimental import pallas as pl
from jax.experimental.pallas import tpu as pltpu
from jax.experimental.pallas import tpu_sc as plsc

_NB = 64


@functools.partial(jax.jit, static_argnames=("b", "t"))
def _lookup_t(xq, w2, *, b, t):
    mesh = plsc.VectorSubcoreMesh(core_axis_name="c", subcore_axis_name="s")
    info = plsc.get_sparse_core_info()
    nc = info.num_cores
    nw = nc * info.num_subcores
    n = b * t
    bt = b // 128
    chunks_per_j = bt // _NB
    sub_o = _NB * 512

    @functools.partial(
        pl.kernel,
        mesh=mesh,
        out_type=jax.ShapeDtypeStruct((4 * n,), jnp.float32),
        scratch_types=[
            pltpu.VMEM((_NB, 128), jnp.int32),
            pltpu.VMEM((_NB, 128), jnp.int32),
            pltpu.VMEM((sub_o,), jnp.float32),
            pltpu.VMEM((sub_o,), jnp.float32),
            pltpu.VMEM((16, 16), jnp.float32),
            pltpu.SemaphoreType.DMA,
            pltpu.SemaphoreType.DMA,
            pltpu.SemaphoreType.DMA,
        ],
        compiler_params=pltpu.CompilerParams(needs_layout_passes=False),
    )
    def k(
        x_hbm,
        w_hbm,
        out_hbm,
        x_v0,
        x_v1,
        out_v0,
        out_v1,
        tbl_v,
        in_sem,
        out_sem0,
        out_sem1,
    ):
        wid = lax.axis_index("s") * nc + lax.axis_index("c")
        n_chunks = t * chunks_per_j
        q_lo = (wid * n_chunks) // nw
        q_hi = ((wid + 1) * n_chunks) // nw
        steps = q_hi - q_lo

        pltpu.sync_copy(w_hbm, tbl_v)
        io = lax.iota(jnp.int32, 16)
        x_vs = (x_v0, x_v1)
        out_vs = (out_v0, out_v1)
        out_sems = (out_sem0, out_sem1)

        def step_coords(step):
            q = q_lo + step
            j = q // chunks_per_j
            b0 = (q % chunks_per_j) * _NB
            return j, b0

        def start_in(step, slot):
            j, b0 = step_coords(step)
            pltpu.async_copy(
                x_hbm.at[j // 8, pl.ds(pl.multiple_of(b0, 8), _NB), j % 8],
                x_vs[slot],
                in_sem,
            )

        def wait_in(slot):
            pltpu.make_async_copy(
                x_hbm.at[0, pl.ds(0, _NB), 0], x_vs[slot], in_sem
            ).wait()

        def wait_out(slot):
            pltpu.make_async_copy(
                out_vs[slot], out_hbm.at[pl.ds(0, sub_o)], out_sems[slot]
            ).wait()

        start_in(0, 0)

        def pair_body(it2, _):
            for slot in (0, 1):
                step = 2 * it2 + slot
                x_v = x_vs[slot]
                out_v = out_vs[slot]

                @pl.when(step < steps)
                def _():
                    wait_in(slot)

                    @pl.when(step + 1 < steps)
                    def _():
                        start_in(step + 1, 1 - slot)

                    @pl.when(it2 >= 1)
                    def _():
                        wait_out(slot)

                    @plsc.parallel_loop(0, _NB, unroll=1)
                    def blk_body(blk):
                        ob = blk * 512
                        tix = [
                            lax.shift_left(x_v[blk, pl.ds(16 * v, 16)], 2)
                            for v in range(8)
                        ]
                        for kk in range(4):
                            os = [
                                plsc.load_gather(tbl_v, [tix[v] + kk, io])
                                for v in range(8)
                            ]
                            for v in range(8):
                                out_v[pl.ds(ob + 128 * kk + 16 * v, 16)] = (
                                    os[v]
                                )

                    j, b0 = step_coords(step)
                    off = pl.multiple_of((j * bt + b0) * 512, 8)
                    pltpu.async_copy(
                        out_v, out_hbm.at[pl.ds(off, sub_o)], out_sems[slot]
                    )
            return 0

        lax.fori_loop(0, (steps + 1) // 2, pair_body, 0)
        wait_out(0)
        wait_out(1)

    return k(xq, w2)


def kernel(x, emb_weight):
    b, t = x.shape
    assert b % 128 == 0 and t % 8 == 0
    xq = (
        x.astype(jnp.int32)
        .reshape(b // 128, 128, t // 8, 8)
        .transpose(2, 0, 3, 1)
    )
    w_flat = emb_weight.reshape(16).astype(jnp.float32)
    w2 = jnp.tile(w_flat[:, None], (1, 16))

    y_flat = _lookup_t(xq, w2, b=b, t=t)
    z = y_flat.reshape(t, b // 128, 4, 128)
    return z.transpose(1, 3, 0, 2).reshape(b, t, 4)
